# Initial kernel scaffold; baseline (speedup 1.0000x reference)
import jax
import jax.numpy as jnp
from jax.experimental import pallas as pl


def kernel(x, edge_index, edge_type, comp, bases, root, bias):
    raise NotImplementedError("write your pallas kernel here")


# speedup vs baseline: 2.2243x; 2.2243x over previous
import functools

import jax
import jax.numpy as jnp
from jax import lax
from jax.experimental import pallas as pl
from jax.experimental.pallas import tpu as pltpu
from jax.experimental.pallas import tpu_sc as plsc

N = 10000
E = 320000
R = 16
NB = 8
DIN = 128
DOUT = 128

NC = 2
NS = 16
EPC = E // NC
EPT = EPC // NS
CEPT = E // NS
CH = 80
RN = R * N
RNT = RN // NS
NG = N // 8


def _w_body(comp_ref, bases_ref, w_ref):
    w_ref[...] = jnp.dot(comp_ref[...], bases_ref[...],
                         preferred_element_type=jnp.float32)


def _make_w(comp, bases2):
    return pl.pallas_call(
        _w_body,
        out_shape=jax.ShapeDtypeStruct((R, DIN * DOUT), jnp.float32),
    )(comp, bases2)


BN = 1000
NBLK = N // BN


def _z_body(x_ref, w_ref, root_ref, bias_ref, z_ref, rp_ref):
    r = pl.program_id(1)
    xb = x_ref[...]
    z_ref[0] = jnp.dot(xb, w_ref[0], preferred_element_type=jnp.float32)

    @pl.when(r == 0)
    def _():
        rp_ref[...] = (jnp.dot(xb, root_ref[...],
                               preferred_element_type=jnp.float32)
                       + bias_ref[...])


def _make_z(x, w, root, bias2):
    return pl.pallas_call(
        _z_body,
        grid=(NBLK, R),
        in_specs=[
            pl.BlockSpec((BN, DIN), lambda nb, r: (nb, 0)),
            pl.BlockSpec((1, DIN, DOUT), lambda nb, r: (r, 0, 0)),
            pl.BlockSpec((DIN, DOUT), lambda nb, r: (0, 0)),
            pl.BlockSpec((1, DOUT), lambda nb, r: (0, 0)),
        ],
        out_specs=[
            pl.BlockSpec((1, BN, DOUT), lambda nb, r: (r, nb, 0)),
            pl.BlockSpec((BN, DOUT), lambda nb, r: (nb, 0)),
        ],
        out_shape=[
            jax.ShapeDtypeStruct((R, N, DOUT), jnp.float32),
            jax.ShapeDtypeStruct((N, DOUT), jnp.float32),
        ],
    )(x, w, root, bias2)


_sc_mesh = plsc.VectorSubcoreMesh(core_axis_name="c", subcore_axis_name="s")


@functools.partial(
    pl.kernel,
    out_type=jax.ShapeDtypeStruct((NC, N, DOUT), jnp.float32),
    mesh=_sc_mesh,
    compiler_params=pltpu.CompilerParams(needs_layout_passes=False),
    scratch_types=[
        pltpu.VMEM_SHARED((RN,), jnp.float32),
        pltpu.VMEM_SHARED((N, DOUT), jnp.float32),
        pltpu.VMEM((CH,), jnp.int32),
        pltpu.VMEM((CH,), jnp.int32),
        pltpu.VMEM((CH,), jnp.int32),
        pltpu.VMEM((CH,), jnp.int32),
        pltpu.VMEM((CH,), jnp.int32),
        pltpu.VMEM((CH,), jnp.float32),
        pltpu.VMEM((CH,), jnp.float32),
        pltpu.VMEM((CH, DOUT), jnp.float32),
        pltpu.VMEM((RNT,), jnp.float32),
        pltpu.SemaphoreType.DMA,
    ],
)
def _sc_agg(src_hbm, dst_hbm, et_hbm, z_hbm, out_hbm,
            cnt_sh, agg_sh, srcb, dstb, typb, gb, segb, wb, onesb,
            rowsb, invb, sem):
    cid = lax.axis_index("c")
    sid = lax.axis_index("s")

    def _z16(i, _):
        invb[pl.ds(i * 16, 16)] = jnp.zeros((16,), jnp.float32)
        return 0
    lax.fori_loop(0, RNT // 16, _z16, 0)

    def _zrow(i, _):
        for k in range(DOUT // 16):
            rowsb[i, pl.ds(k * 16, 16)] = jnp.zeros((16,), jnp.float32)
        return 0
    lax.fori_loop(0, CH, _zrow, 0)

    for i in range(CH // 16):
        onesb[pl.ds(i * 16, 16)] = jnp.ones((16,), jnp.float32)

    pltpu.sync_copy(invb, cnt_sh.at[pl.ds(sid * RNT, RNT)])

    def _zagg(k, _):
        g = k * NS + sid

        @pl.when(g < NG)
        def _():
            pltpu.sync_copy(rowsb.at[pl.ds(0, 8)],
                            agg_sh.at[pl.ds(g * 8, 8)])
        return 0
    lax.fori_loop(0, (NG + NS - 1) // NS, _zagg, 0)
    plsc.subcore_barrier()

    def _count_chunk(c, _):
        base = sid * CEPT + c * CH
        pltpu.sync_copy(dst_hbm.at[pl.ds(base, CH)], dstb)
        pltpu.sync_copy(et_hbm.at[pl.ds(base, CH)], typb)

        def _mk(i, _):
            sl = pl.ds(i * 16, 16)
            segb[sl] = typb[sl] * N + dstb[sl]
            return 0
        lax.fori_loop(0, CH // 16, _mk, 0)
        pltpu.sync_copy(onesb, cnt_sh.at[segb], add=True)
        return 0
    lax.fori_loop(0, CEPT // CH, _count_chunk, 0)
    plsc.subcore_barrier()

    pltpu.sync_copy(cnt_sh.at[pl.ds(sid * RNT, RNT)], invb)

    def _inv16(i, _):
        sl = pl.ds(i * 16, 16)
        invb[sl] = 1.0 / jnp.maximum(invb[sl], 1.0)
        return 0
    lax.fori_loop(0, RNT // 16, _inv16, 0)
    pltpu.sync_copy(invb, cnt_sh.at[pl.ds(sid * RNT, RNT)])
    plsc.subcore_barrier()

    def _main_chunk(c, _):
        base = cid * EPC + sid * EPT + c * CH
        pltpu.sync_copy(src_hbm.at[pl.ds(base, CH)], srcb)
        pltpu.sync_copy(dst_hbm.at[pl.ds(base, CH)], dstb)
        pltpu.sync_copy(et_hbm.at[pl.ds(base, CH)], typb)

        def _mk(i, _):
            sl = pl.ds(i * 16, 16)
            t = typb[sl]
            gb[sl] = t * N + srcb[sl]
            segb[sl] = t * N + dstb[sl]
            return 0
        lax.fori_loop(0, CH // 16, _mk, 0)

        pltpu.sync_copy(cnt_sh.at[segb], wb)
        pltpu.async_copy(z_hbm.at[gb], rowsb, sem).wait()

        def _scale(j, _):
            wsp = plsc.load_gather(wb, [jnp.full((16,), j, jnp.int32)])
            for k in range(DOUT // 16):
                sl = pl.ds(k * 16, 16)
                rowsb[j, sl] = rowsb[j, sl] * wsp
            return 0
        lax.fori_loop(0, CH, _scale, 0)

        pltpu.sync_copy(rowsb, agg_sh.at[dstb], add=True)
        return 0
    lax.fori_loop(0, EPT // CH, _main_chunk, 0)
    plsc.subcore_barrier()

    def _wout(k, _):
        g = k * NS + sid

        @pl.when(g < NG)
        def _():
            pltpu.sync_copy(agg_sh.at[pl.ds(g * 8, 8)],
                            out_hbm.at[cid, pl.ds(g * 8, 8)])
        return 0
    lax.fori_loop(0, (NG + NS - 1) // NS, _wout, 0)


def _fin_body(parts_ref, rp_ref, out_ref):
    out_ref[...] = parts_ref[0] + parts_ref[1] + rp_ref[...]


def _fin(parts, rootp):
    return pl.pallas_call(
        _fin_body,
        grid=(NBLK,),
        in_specs=[
            pl.BlockSpec((NC, BN, DOUT), lambda nb: (0, nb, 0)),
            pl.BlockSpec((BN, DOUT), lambda nb: (nb, 0)),
        ],
        out_specs=pl.BlockSpec((BN, DOUT), lambda nb: (nb, 0)),
        out_shape=jax.ShapeDtypeStruct((N, DOUT), jnp.float32),
    )(parts, rootp)


def kernel(x, edge_index, edge_type, comp, bases, root, bias):
    bases2 = bases.reshape(NB, DIN * DOUT)
    w = _make_w(comp, bases2).reshape(R, DIN, DOUT)
    z, rootp = _make_z(x, w, root, bias.reshape(1, DOUT))
    zf = z.reshape(RN, DOUT)
    parts = _sc_agg(edge_index[0], edge_index[1], edge_type, zf)
    return _fin(parts, rootp)


# speedup vs baseline: 3.5147x; 1.5801x over previous
import functools

import jax
import jax.numpy as jnp
from jax import lax
from jax.experimental import pallas as pl
from jax.experimental.pallas import tpu as pltpu
from jax.experimental.pallas import tpu_sc as plsc

N = 10000
E = 320000
R = 16
NB = 8
DIN = 128
DOUT = 128

NC = 2
NS = 16
EPC = E // NC
EPT = EPC // NS
CEPT = E // NS
CH = 80
RN = R * N
RNT = RN // NS
NG = N // 8
IB = 2000


def _w_body(comp_ref, bases_ref, w_ref):
    w_ref[...] = jnp.dot(comp_ref[...], bases_ref[...],
                         preferred_element_type=jnp.float32)


def _make_w(comp, bases2):
    return pl.pallas_call(
        _w_body,
        out_shape=jax.ShapeDtypeStruct((R, DIN * DOUT), jnp.float32),
    )(comp, bases2)


BN = 1000
NBLK = N // BN


def _z_body(x_ref, w_ref, root_ref, bias_ref, z_ref, rp_ref):
    r = pl.program_id(1)
    xb = x_ref[...]
    z_ref[0] = jnp.dot(xb, w_ref[0], preferred_element_type=jnp.float32)

    @pl.when(r == 0)
    def _():
        rp_ref[...] = (jnp.dot(xb, root_ref[...],
                               preferred_element_type=jnp.float32)
                       + bias_ref[...])


def _make_z(x, w, root, bias2):
    return pl.pallas_call(
        _z_body,
        grid=(NBLK, R),
        in_specs=[
            pl.BlockSpec((BN, DIN), lambda nb, r: (nb, 0)),
            pl.BlockSpec((1, DIN, DOUT), lambda nb, r: (r, 0, 0)),
            pl.BlockSpec((DIN, DOUT), lambda nb, r: (0, 0)),
            pl.BlockSpec((1, DOUT), lambda nb, r: (0, 0)),
        ],
        out_specs=[
            pl.BlockSpec((1, BN, DOUT), lambda nb, r: (r, nb, 0)),
            pl.BlockSpec((BN, DOUT), lambda nb, r: (nb, 0)),
        ],
        out_shape=[
            jax.ShapeDtypeStruct((R, N, DOUT), jnp.float32),
            jax.ShapeDtypeStruct((N, DOUT), jnp.float32),
        ],
    )(x, w, root, bias2)


_sc_mesh = plsc.VectorSubcoreMesh(core_axis_name="c", subcore_axis_name="s")


@functools.partial(
    pl.kernel,
    out_type=jax.ShapeDtypeStruct((NC, N, DOUT), jnp.float32),
    mesh=_sc_mesh,
    compiler_params=pltpu.CompilerParams(needs_layout_passes=False),
    scratch_types=[
        pltpu.VMEM_SHARED((RN,), jnp.float32),
        pltpu.VMEM_SHARED((N, DOUT), jnp.float32),
        pltpu.VMEM((CH,), jnp.int32),
        pltpu.VMEM((CH,), jnp.int32),
        pltpu.VMEM((CH,), jnp.int32),
        pltpu.VMEM((CH,), jnp.int32),
        pltpu.VMEM((CH,), jnp.int32),
        pltpu.VMEM((CH,), jnp.float32),
        pltpu.VMEM((CH,), jnp.float32),
        pltpu.VMEM((CH,), jnp.int32),
        pltpu.VMEM((CH,), jnp.float32),
        pltpu.VMEM((CH, DOUT), jnp.float32),
        pltpu.VMEM((CH, DOUT), jnp.float32),
        pltpu.VMEM((IB,), jnp.float32),
        pltpu.SemaphoreType.DMA,
        pltpu.SemaphoreType.DMA,
        pltpu.SemaphoreType.DMA,
        pltpu.SemaphoreType.DMA,
    ],
)
def _sc_agg(pk_hbm, z_hbm, out_hbm,
            cnt_sh, agg_sh, pb,
            gb0, gb1, db0, db1, wb0, wb1, segb, onesb,
            rows0, rows1, invb, sg0, sg1, ss0, ss1):
    cid = lax.axis_index("c")
    sid = lax.axis_index("s")
    m14 = jnp.full((16,), 16383, jnp.int32)

    def _z16(i, _):
        invb[pl.ds(i * 16, 16)] = jnp.zeros((16,), jnp.float32)
        return 0
    lax.fori_loop(0, IB // 16, _z16, 0)

    def _zrow(i, _):
        for k in range(DOUT // 16):
            rows0[i, pl.ds(k * 16, 16)] = jnp.zeros((16,), jnp.float32)
        return 0
    lax.fori_loop(0, CH, _zrow, 0)

    for i in range(CH // 16):
        onesb[pl.ds(i * 16, 16)] = jnp.ones((16,), jnp.float32)

    for h in range(RNT // IB):
        pltpu.sync_copy(invb, cnt_sh.at[pl.ds(sid * RNT + h * IB, IB)])

    def _zagg(k, _):
        g = k * NS + sid

        @pl.when(g < NG)
        def _():
            pltpu.sync_copy(rows0.at[pl.ds(0, 8)],
                            agg_sh.at[pl.ds(g * 8, 8)])
        return 0
    lax.fori_loop(0, (NG + NS - 1) // NS, _zagg, 0)
    plsc.subcore_barrier()

    def _count_chunk(c, _):
        pltpu.sync_copy(pk_hbm.at[pl.ds(sid * CEPT + c * CH, CH)], pb)

        def _mk(i, _):
            sl = pl.ds(i * 16, 16)
            w_ = pb[sl]
            t = lax.shift_right_logical(w_, 28)
            segb[sl] = t * N + (w_ & m14)
            return 0
        lax.fori_loop(0, CH // 16, _mk, 0)
        pltpu.sync_copy(onesb, cnt_sh.at[segb], add=True)
        return 0
    lax.fori_loop(0, CEPT // CH, _count_chunk, 0)
    plsc.subcore_barrier()

    for h in range(RNT // IB):
        off = sid * RNT + h * IB
        pltpu.sync_copy(cnt_sh.at[pl.ds(off, IB)], invb)

        def _inv16(i, _):
            sl = pl.ds(i * 16, 16)
            invb[sl] = 1.0 / jnp.maximum(invb[sl], 1.0)
            return 0
        lax.fori_loop(0, IB // 16, _inv16, 0)
        pltpu.sync_copy(invb, cnt_sh.at[pl.ds(off, IB)])
    plsc.subcore_barrier()

    ebase = cid * EPC + sid * EPT

    def _prep_fire(c, gbp, dbp, wbp, rowsp, sgp, ssp, wait_scatter):
        if wait_scatter:
            pltpu.make_async_copy(rowsp, agg_sh.at[dbp], ssp).wait()

        pltpu.sync_copy(pk_hbm.at[pl.ds(ebase + c * CH, CH)], pb)

        def _mk(i, _):
            sl = pl.ds(i * 16, 16)
            w_ = pb[sl]
            t = lax.shift_right_logical(w_, 28)
            d = w_ & m14
            gbp[sl] = t * N + (lax.shift_right_logical(w_, 14) & m14)
            dbp[sl] = d
            segb[sl] = t * N + d
            return 0
        lax.fori_loop(0, CH // 16, _mk, 0)
        pltpu.sync_copy(cnt_sh.at[segb], wbp)
        pltpu.async_copy(z_hbm.at[gbp], rowsp, sgp)

    def _consume(gbp, dbp, wbp, rowsp, sgp, ssp):
        pltpu.make_async_copy(z_hbm.at[gbp], rowsp, sgp).wait()

        def _scale(j, _):
            wsp = plsc.load_gather(wbp, [jnp.full((16,), j, jnp.int32)])
            for k in range(DOUT // 16):
                sl = pl.ds(k * 16, 16)
                rowsp[j, sl] = rowsp[j, sl] * wsp
            return 0
        lax.fori_loop(0, CH, _scale, 0)
        pltpu.async_copy(rowsp, agg_sh.at[dbp], ssp, add=True)

    slot0 = (gb0, db0, wb0, rows0, sg0, ss0)
    slot1 = (gb1, db1, wb1, rows1, sg1, ss1)
    NCH = EPT // CH

    _prep_fire(0, *slot0, wait_scatter=False)
    _prep_fire(1, *slot1, wait_scatter=False)

    def _pipe(k, _):
        _consume(*slot0)
        _prep_fire(2 * k + 2, *slot0, wait_scatter=True)
        _consume(*slot1)

        @pl.when(2 * k + 3 < NCH)
        def _():
            _prep_fire(2 * k + 3, *slot1, wait_scatter=True)
        return 0
    lax.fori_loop(0, (NCH - 1) // 2, _pipe, 0)
    _consume(*slot0)
    pltpu.make_async_copy(rows0, agg_sh.at[db0], ss0).wait()
    pltpu.make_async_copy(rows1, agg_sh.at[db1], ss1).wait()
    plsc.subcore_barrier()

    def _wout(k, _):
        g = k * NS + sid

        @pl.when(g < NG)
        def _():
            pltpu.sync_copy(agg_sh.at[pl.ds(g * 8, 8)],
                            out_hbm.at[cid, pl.ds(g * 8, 8)])
        return 0
    lax.fori_loop(0, (NG + NS - 1) // NS, _wout, 0)


def _fin_body(parts_ref, rp_ref, out_ref):
    out_ref[...] = parts_ref[0] + parts_ref[1] + rp_ref[...]


def _fin(parts, rootp):
    return pl.pallas_call(
        _fin_body,
        grid=(NBLK,),
        in_specs=[
            pl.BlockSpec((NC, BN, DOUT), lambda nb: (0, nb, 0)),
            pl.BlockSpec((BN, DOUT), lambda nb: (nb, 0)),
        ],
        out_specs=pl.BlockSpec((BN, DOUT), lambda nb: (nb, 0)),
        out_shape=jax.ShapeDtypeStruct((N, DOUT), jnp.float32),
    )(parts, rootp)


def kernel(x, edge_index, edge_type, comp, bases, root, bias):
    bases2 = bases.reshape(NB, DIN * DOUT)
    w = _make_w(comp, bases2).reshape(R, DIN, DOUT)
    z, rootp = _make_z(x, w, root, bias.reshape(1, DOUT))
    zf = z.reshape(RN, DOUT)
    packed = ((edge_type << 28) | (edge_index[0] << 14) | edge_index[1])
    parts = _sc_agg(packed.astype(jnp.int32), zf)
    return _fin(parts, rootp)


# speedup vs baseline: 4.4336x; 1.2614x over previous
import functools

import jax
import jax.numpy as jnp
from jax import lax
from jax.experimental import pallas as pl
from jax.experimental.pallas import tpu as pltpu
from jax.experimental.pallas import tpu_sc as plsc

N = 10000
E = 320000
R = 16
NB = 8
DIN = 128
DOUT = 128

NC = 2
NS = 16
EPC = E // NC
EPT = EPC // NS
CEPT = E // NS
CH = 80
RN = R * N
RNT = RN // NS
NG = N // 8
IB = 2000


def _w_body(comp_ref, bases_ref, w_ref):
    w_ref[...] = jnp.dot(comp_ref[...], bases_ref[...],
                         preferred_element_type=jnp.float32)


def _make_w(comp, bases2):
    return pl.pallas_call(
        _w_body,
        out_shape=jax.ShapeDtypeStruct((R, DIN * DOUT), jnp.float32),
    )(comp, bases2)


BN = 1000
NBLK = N // BN


def _z_body(x_ref, w_ref, root_ref, bias_ref, z_ref, rp_ref):
    r = pl.program_id(1)
    xb = x_ref[...]
    z_ref[0] = jnp.dot(xb, w_ref[0], preferred_element_type=jnp.float32)

    @pl.when(r == 0)
    def _():
        rp_ref[...] = (jnp.dot(xb, root_ref[...],
                               preferred_element_type=jnp.float32)
                       + bias_ref[...])


def _make_z(x, w, root, bias2):
    return pl.pallas_call(
        _z_body,
        grid=(NBLK, R),
        in_specs=[
            pl.BlockSpec((BN, DIN), lambda nb, r: (nb, 0)),
            pl.BlockSpec((1, DIN, DOUT), lambda nb, r: (r, 0, 0)),
            pl.BlockSpec((DIN, DOUT), lambda nb, r: (0, 0)),
            pl.BlockSpec((1, DOUT), lambda nb, r: (0, 0)),
        ],
        out_specs=[
            pl.BlockSpec((1, BN, DOUT), lambda nb, r: (r, nb, 0)),
            pl.BlockSpec((BN, DOUT), lambda nb, r: (nb, 0)),
        ],
        out_shape=[
            jax.ShapeDtypeStruct((R, N, DOUT), jnp.float32),
            jax.ShapeDtypeStruct((N, DOUT), jnp.float32),
        ],
    )(x, w, root, bias2)


_sc_mesh = plsc.VectorSubcoreMesh(core_axis_name="c", subcore_axis_name="s")


@functools.partial(
    pl.kernel,
    out_type=jax.ShapeDtypeStruct((NC, N, DOUT), jnp.float32),
    mesh=_sc_mesh,
    compiler_params=pltpu.CompilerParams(needs_layout_passes=False),
    scratch_types=[
        pltpu.VMEM_SHARED((RN,), jnp.float32),
        pltpu.VMEM_SHARED((N, DOUT), jnp.float32),
        pltpu.VMEM((CH,), jnp.int32),
        pltpu.VMEM((CH,), jnp.int32),
        pltpu.VMEM((CH,), jnp.int32),
        pltpu.VMEM((CH,), jnp.int32),
        pltpu.VMEM((CH,), jnp.int32),
        pltpu.VMEM((CH,), jnp.float32),
        pltpu.VMEM((CH,), jnp.float32),
        pltpu.VMEM((CH,), jnp.int32),
        pltpu.VMEM((CH,), jnp.float32),
        pltpu.VMEM((CH, DOUT), jnp.float32),
        pltpu.VMEM((CH, DOUT), jnp.float32),
        pltpu.VMEM((IB,), jnp.float32),
        pltpu.SemaphoreType.DMA,
        pltpu.SemaphoreType.DMA,
        pltpu.SemaphoreType.DMA,
        pltpu.SemaphoreType.DMA,
    ],
)
def _sc_agg(pk_hbm, z_hbm, out_hbm,
            cnt_sh, agg_sh, pb,
            gb0, gb1, db0, db1, wb0, wb1, segb, onesb,
            rows0, rows1, invb, sg0, sg1, ss0, ss1):
    cid = lax.axis_index("c")
    sid = lax.axis_index("s")
    m14 = jnp.full((16,), 16383, jnp.int32)

    def _z16(i, _):
        invb[pl.ds(i * 16, 16)] = jnp.zeros((16,), jnp.float32)
        return 0
    lax.fori_loop(0, IB // 16, _z16, 0)

    def _zrow(i, _):
        for k in range(DOUT // 16):
            rows0[i, pl.ds(k * 16, 16)] = jnp.zeros((16,), jnp.float32)
        return 0
    lax.fori_loop(0, CH, _zrow, 0)

    for i in range(CH // 16):
        onesb[pl.ds(i * 16, 16)] = jnp.ones((16,), jnp.float32)

    for h in range(RNT // IB):
        pltpu.sync_copy(invb, cnt_sh.at[pl.ds(sid * RNT + h * IB, IB)])

    def _zagg(k, _):
        g = k * NS + sid

        @pl.when(g < NG)
        def _():
            pltpu.sync_copy(rows0.at[pl.ds(0, 8)],
                            agg_sh.at[pl.ds(g * 8, 8)])
        return 0
    lax.fori_loop(0, (NG + NS - 1) // NS, _zagg, 0)
    plsc.subcore_barrier()

    def _cload(c, pbp, sgp):
        pltpu.async_copy(pk_hbm.at[pl.ds(sid * CEPT + c * CH, CH)], pbp, sgp)

    def _ccount(c, pbp, sbp, sgp, ssp, wait_scatter):
        pltpu.make_async_copy(pk_hbm.at[pl.ds(0, CH)], pbp, sgp).wait()
        if wait_scatter:
            pltpu.make_async_copy(onesb, cnt_sh.at[sbp], ssp).wait()

        @plsc.parallel_loop(0, CH // 16, 1, unroll=5)
        def _mk(i):
            sl = pl.ds(i * 16, 16)
            w_ = pbp[sl]
            t = lax.shift_right_logical(w_, 28)
            sbp[sl] = t * N + (w_ & m14)
        pltpu.async_copy(onesb, cnt_sh.at[sbp], ssp, add=True)

    cslot0 = (gb0, db0, sg0, ss0)
    cslot1 = (gb1, db1, sg1, ss1)
    CNCH = CEPT // CH

    _cload(0, gb0, sg0)
    _cload(1, gb1, sg1)
    _ccount(0, *cslot0, wait_scatter=False)
    _cload(2, gb0, sg0)
    _ccount(1, *cslot1, wait_scatter=False)
    _cload(3, gb1, sg1)

    def _cpipe(k, _):
        _ccount(2 * k + 2, *cslot0, wait_scatter=True)

        @pl.when(2 * k + 4 < CNCH)
        def _():
            _cload(2 * k + 4, gb0, sg0)
        _ccount(2 * k + 3, *cslot1, wait_scatter=True)

        @pl.when(2 * k + 5 < CNCH)
        def _():
            _cload(2 * k + 5, gb1, sg1)
        return 0
    lax.fori_loop(0, CNCH // 2 - 1, _cpipe, 0)
    pltpu.make_async_copy(onesb, cnt_sh.at[db0], ss0).wait()
    pltpu.make_async_copy(onesb, cnt_sh.at[db1], ss1).wait()
    plsc.subcore_barrier()

    for h in range(RNT // IB):
        off = sid * RNT + h * IB
        pltpu.sync_copy(cnt_sh.at[pl.ds(off, IB)], invb)

        def _inv16(i, _):
            sl = pl.ds(i * 16, 16)
            invb[sl] = 1.0 / jnp.maximum(invb[sl], 1.0)
            return 0
        lax.fori_loop(0, IB // 16, _inv16, 0)
        pltpu.sync_copy(invb, cnt_sh.at[pl.ds(off, IB)])
    plsc.subcore_barrier()

    ebase = cid * EPC + sid * EPT

    def _prep_fire(c, gbp, dbp, wbp, rowsp, sgp, ssp, wait_scatter):
        if wait_scatter:
            pltpu.make_async_copy(rowsp, agg_sh.at[dbp], ssp).wait()

        pltpu.sync_copy(pk_hbm.at[pl.ds(ebase + c * CH, CH)], pb)

        @plsc.parallel_loop(0, CH // 16, 1, unroll=5)
        def _mk(i):
            sl = pl.ds(i * 16, 16)
            w_ = pb[sl]
            t = lax.shift_right_logical(w_, 28)
            d = w_ & m14
            gbp[sl] = t * N + (lax.shift_right_logical(w_, 14) & m14)
            dbp[sl] = d
            segb[sl] = t * N + d
        pltpu.sync_copy(cnt_sh.at[segb], wbp)
        pltpu.async_copy(z_hbm.at[gbp], rowsp, sgp)

    def _consume(gbp, dbp, wbp, rowsp, sgp, ssp):
        pltpu.make_async_copy(z_hbm.at[gbp], rowsp, sgp).wait()

        @plsc.parallel_loop(0, CH, 1, unroll=4)
        def _scale(j):
            wsp = plsc.load_gather(wbp, [jnp.full((16,), j, jnp.int32)])
            for k in range(DOUT // 16):
                sl = pl.ds(k * 16, 16)
                rowsp[j, sl] = rowsp[j, sl] * wsp
        pltpu.async_copy(rowsp, agg_sh.at[dbp], ssp, add=True)

    slot0 = (gb0, db0, wb0, rows0, sg0, ss0)
    slot1 = (gb1, db1, wb1, rows1, sg1, ss1)
    NCH = EPT // CH

    _prep_fire(0, *slot0, wait_scatter=False)
    _prep_fire(1, *slot1, wait_scatter=False)

    def _pipe(k, _):
        _consume(*slot0)
        _prep_fire(2 * k + 2, *slot0, wait_scatter=True)
        _consume(*slot1)

        @pl.when(2 * k + 3 < NCH)
        def _():
            _prep_fire(2 * k + 3, *slot1, wait_scatter=True)
        return 0
    lax.fori_loop(0, (NCH - 1) // 2, _pipe, 0)
    _consume(*slot0)
    pltpu.make_async_copy(rows0, agg_sh.at[db0], ss0).wait()
    pltpu.make_async_copy(rows1, agg_sh.at[db1], ss1).wait()
    plsc.subcore_barrier()

    def _wout(k, _):
        g = k * NS + sid

        @pl.when(g < NG)
        def _():
            pltpu.sync_copy(agg_sh.at[pl.ds(g * 8, 8)],
                            out_hbm.at[cid, pl.ds(g * 8, 8)])
        return 0
    lax.fori_loop(0, (NG + NS - 1) // NS, _wout, 0)


def _fin_body(parts_ref, rp_ref, out_ref):
    out_ref[...] = parts_ref[0] + parts_ref[1] + rp_ref[...]


def _fin(parts, rootp):
    return pl.pallas_call(
        _fin_body,
        grid=(NBLK,),
        in_specs=[
            pl.BlockSpec((NC, BN, DOUT), lambda nb: (0, nb, 0)),
            pl.BlockSpec((BN, DOUT), lambda nb: (nb, 0)),
        ],
        out_specs=pl.BlockSpec((BN, DOUT), lambda nb: (nb, 0)),
        out_shape=jax.ShapeDtypeStruct((N, DOUT), jnp.float32),
    )(parts, rootp)


def kernel(x, edge_index, edge_type, comp, bases, root, bias):
    bases2 = bases.reshape(NB, DIN * DOUT)
    w = _make_w(comp, bases2).reshape(R, DIN, DOUT)
    z, rootp = _make_z(x, w, root, bias.reshape(1, DOUT))
    zf = z.reshape(RN, DOUT)
    packed = ((edge_type << 28) | (edge_index[0] << 14) | edge_index[1])
    parts = _sc_agg(packed.astype(jnp.int32), zf)
    return _fin(parts, rootp)


# speedup vs baseline: 4.5754x; 1.0320x over previous
import functools

import jax
import jax.numpy as jnp
from jax import lax
from jax.experimental import pallas as pl
from jax.experimental.pallas import tpu as pltpu
from jax.experimental.pallas import tpu_sc as plsc

N = 10000
E = 320000
R = 16
NB = 8
DIN = 128
DOUT = 128

NC = 2
NS = 16
EPC = E // NC
EPT = EPC // NS
CEPT = E // NS
CH = 80
RN = R * N
RNT = RN // NS
NG = N // 8
IB = 2000


def _w_body(comp_ref, bases_ref, w_ref):
    w_ref[...] = jnp.dot(comp_ref[...], bases_ref[...],
                         preferred_element_type=jnp.float32)


def _make_w(comp, bases2):
    return pl.pallas_call(
        _w_body,
        out_shape=jax.ShapeDtypeStruct((R, DIN * DOUT), jnp.float32),
    )(comp, bases2)


BN = 1000
NBLK = N // BN


def _z_body(x_ref, w_ref, root_ref, bias_ref, z_ref, rp_ref):
    r = pl.program_id(1)
    xb = x_ref[...]
    z_ref[0] = jnp.dot(xb, w_ref[0], preferred_element_type=jnp.float32)

    @pl.when(r == 0)
    def _():
        rp_ref[...] = (jnp.dot(xb, root_ref[...],
                               preferred_element_type=jnp.float32)
                       + bias_ref[...])


def _make_z(x, w, root, bias2):
    return pl.pallas_call(
        _z_body,
        grid=(NBLK, R),
        in_specs=[
            pl.BlockSpec((BN, DIN), lambda nb, r: (nb, 0)),
            pl.BlockSpec((1, DIN, DOUT), lambda nb, r: (r, 0, 0)),
            pl.BlockSpec((DIN, DOUT), lambda nb, r: (0, 0)),
            pl.BlockSpec((1, DOUT), lambda nb, r: (0, 0)),
        ],
        out_specs=[
            pl.BlockSpec((1, BN, DOUT), lambda nb, r: (r, nb, 0)),
            pl.BlockSpec((BN, DOUT), lambda nb, r: (nb, 0)),
        ],
        out_shape=[
            jax.ShapeDtypeStruct((R, N, DOUT), jnp.float32),
            jax.ShapeDtypeStruct((N, DOUT), jnp.float32),
        ],
    )(x, w, root, bias2)


_sc_mesh = plsc.VectorSubcoreMesh(core_axis_name="c", subcore_axis_name="s")


@functools.partial(
    pl.kernel,
    out_type=jax.ShapeDtypeStruct((NC, N, DOUT), jnp.float32),
    mesh=_sc_mesh,
    compiler_params=pltpu.CompilerParams(needs_layout_passes=False),
    scratch_types=[
        pltpu.VMEM_SHARED((RN,), jnp.float32),
        pltpu.VMEM_SHARED((N, DOUT), jnp.float32),
        pltpu.VMEM((CH,), jnp.int32),
        pltpu.VMEM((CH,), jnp.int32),
        pltpu.VMEM((CH,), jnp.int32),
        pltpu.VMEM((CH,), jnp.int32),
        pltpu.VMEM((CH,), jnp.int32),
        pltpu.VMEM((CH,), jnp.float32),
        pltpu.VMEM((CH,), jnp.float32),
        pltpu.VMEM((CH,), jnp.int32),
        pltpu.VMEM((CH,), jnp.int32),
        pltpu.VMEM((CH,), jnp.float32),
        pltpu.VMEM((CH, DOUT), jnp.float32),
        pltpu.VMEM((CH, DOUT), jnp.float32),
        pltpu.VMEM((IB,), jnp.float32),
        pltpu.SemaphoreType.DMA,
        pltpu.SemaphoreType.DMA,
        pltpu.SemaphoreType.DMA,
        pltpu.SemaphoreType.DMA,
        pltpu.SemaphoreType.DMA,
        pltpu.SemaphoreType.DMA,
    ],
)
def _sc_agg(pk_hbm, z_hbm, out_hbm,
            cnt_sh, agg_sh, pb,
            gb0, gb1, db0, db1, wb0, wb1, sb0, sb1, onesb,
            rows0, rows1, invb, sg0, sg1, ss0, ss1, sw0, sw1):
    cid = lax.axis_index("c")
    sid = lax.axis_index("s")
    m14 = jnp.full((16,), 16383, jnp.int32)

    def _z16(i, _):
        invb[pl.ds(i * 16, 16)] = jnp.zeros((16,), jnp.float32)
        return 0
    lax.fori_loop(0, IB // 16, _z16, 0)

    def _zrow(i, _):
        for k in range(DOUT // 16):
            rows0[i, pl.ds(k * 16, 16)] = jnp.zeros((16,), jnp.float32)
        return 0
    lax.fori_loop(0, CH, _zrow, 0)

    for i in range(CH // 16):
        onesb[pl.ds(i * 16, 16)] = jnp.ones((16,), jnp.float32)

    for h in range(RNT // IB):
        pltpu.sync_copy(invb, cnt_sh.at[pl.ds(sid * RNT + h * IB, IB)])

    def _zagg(k, _):
        g = k * NS + sid

        @pl.when(g < NG)
        def _():
            pltpu.sync_copy(rows0.at[pl.ds(0, 8)],
                            agg_sh.at[pl.ds(g * 8, 8)])
        return 0
    lax.fori_loop(0, (NG + NS - 1) // NS, _zagg, 0)
    plsc.subcore_barrier()

    def _cload(c, pbp, sgp):
        pltpu.async_copy(pk_hbm.at[pl.ds(sid * CEPT + c * CH, CH)], pbp, sgp)

    def _ccount(c, pbp, sbp, sgp, ssp, wait_scatter):
        pltpu.make_async_copy(pk_hbm.at[pl.ds(0, CH)], pbp, sgp).wait()
        if wait_scatter:
            pltpu.make_async_copy(onesb, cnt_sh.at[sbp], ssp).wait()

        @plsc.parallel_loop(0, CH // 16, 1, unroll=5)
        def _mk(i):
            sl = pl.ds(i * 16, 16)
            w_ = pbp[sl]
            t = lax.shift_right_logical(w_, 28)
            sbp[sl] = t * N + (w_ & m14)
        pltpu.async_copy(onesb, cnt_sh.at[sbp], ssp, add=True)

    cslot0 = (gb0, db0, sg0, ss0)
    cslot1 = (gb1, db1, sg1, ss1)
    CNCH = CEPT // CH

    _cload(0, gb0, sg0)
    _cload(1, gb1, sg1)
    _ccount(0, *cslot0, wait_scatter=False)
    _cload(2, gb0, sg0)
    _ccount(1, *cslot1, wait_scatter=False)
    _cload(3, gb1, sg1)

    def _cpipe(k, _):
        _ccount(2 * k + 2, *cslot0, wait_scatter=True)

        @pl.when(2 * k + 4 < CNCH)
        def _():
            _cload(2 * k + 4, gb0, sg0)
        _ccount(2 * k + 3, *cslot1, wait_scatter=True)

        @pl.when(2 * k + 5 < CNCH)
        def _():
            _cload(2 * k + 5, gb1, sg1)
        return 0
    lax.fori_loop(0, CNCH // 2 - 1, _cpipe, 0)
    pltpu.make_async_copy(onesb, cnt_sh.at[db0], ss0).wait()
    pltpu.make_async_copy(onesb, cnt_sh.at[db1], ss1).wait()
    plsc.subcore_barrier()

    for h in range(RNT // IB):
        off = sid * RNT + h * IB
        pltpu.sync_copy(cnt_sh.at[pl.ds(off, IB)], invb)

        def _inv16(i, _):
            sl = pl.ds(i * 16, 16)
            invb[sl] = 1.0 / jnp.maximum(invb[sl], 1.0)
            return 0
        lax.fori_loop(0, IB // 16, _inv16, 0)
        pltpu.sync_copy(invb, cnt_sh.at[pl.ds(off, IB)])
    plsc.subcore_barrier()

    ebase = cid * EPC + sid * EPT

    def _prep_fire(c, gbp, dbp, wbp, sbp, rowsp, sgp, ssp, swp, wait_scatter):
        if wait_scatter:
            pltpu.make_async_copy(rowsp, agg_sh.at[dbp], ssp).wait()

        pltpu.sync_copy(pk_hbm.at[pl.ds(ebase + c * CH, CH)], pb)

        @plsc.parallel_loop(0, CH // 16, 1, unroll=5)
        def _mk(i):
            sl = pl.ds(i * 16, 16)
            w_ = pb[sl]
            t = lax.shift_right_logical(w_, 28)
            d = w_ & m14
            gbp[sl] = t * N + (lax.shift_right_logical(w_, 14) & m14)
            dbp[sl] = d
            sbp[sl] = t * N + d
        pltpu.async_copy(cnt_sh.at[sbp], wbp, swp)
        pltpu.async_copy(z_hbm.at[gbp], rowsp, sgp)

    def _consume(gbp, dbp, wbp, sbp, rowsp, sgp, ssp, swp):
        pltpu.make_async_copy(cnt_sh.at[sbp], wbp, swp).wait()
        pltpu.make_async_copy(z_hbm.at[gbp], rowsp, sgp).wait()

        @plsc.parallel_loop(0, CH, 1, unroll=4)
        def _scale(j):
            wsp = plsc.load_gather(wbp, [jnp.full((16,), j, jnp.int32)])
            for k in range(DOUT // 16):
                sl = pl.ds(k * 16, 16)
                rowsp[j, sl] = rowsp[j, sl] * wsp
        pltpu.async_copy(rowsp, agg_sh.at[dbp], ssp, add=True)

    slot0 = (gb0, db0, wb0, sb0, rows0, sg0, ss0, sw0)
    slot1 = (gb1, db1, wb1, sb1, rows1, sg1, ss1, sw1)
    NCH = EPT // CH

    _prep_fire(0, *slot0, wait_scatter=False)
    _prep_fire(1, *slot1, wait_scatter=False)

    def _pipe(k, _):
        _consume(*slot0)
        _prep_fire(2 * k + 2, *slot0, wait_scatter=True)
        _consume(*slot1)

        @pl.when(2 * k + 3 < NCH)
        def _():
            _prep_fire(2 * k + 3, *slot1, wait_scatter=True)
        return 0
    lax.fori_loop(0, (NCH - 1) // 2, _pipe, 0)
    _consume(*slot0)
    pltpu.make_async_copy(rows0, agg_sh.at[db0], ss0).wait()
    pltpu.make_async_copy(rows1, agg_sh.at[db1], ss1).wait()
    plsc.subcore_barrier()

    def _wout(k, _):
        g = k * NS + sid

        @pl.when(g < NG)
        def _():
            pltpu.sync_copy(agg_sh.at[pl.ds(g * 8, 8)],
                            out_hbm.at[cid, pl.ds(g * 8, 8)])
        return 0
    lax.fori_loop(0, (NG + NS - 1) // NS, _wout, 0)


def _fin_body(parts_ref, rp_ref, out_ref):
    out_ref[...] = parts_ref[0] + parts_ref[1] + rp_ref[...]


def _fin(parts, rootp):
    return pl.pallas_call(
        _fin_body,
        grid=(NBLK,),
        in_specs=[
            pl.BlockSpec((NC, BN, DOUT), lambda nb: (0, nb, 0)),
            pl.BlockSpec((BN, DOUT), lambda nb: (nb, 0)),
        ],
        out_specs=pl.BlockSpec((BN, DOUT), lambda nb: (nb, 0)),
        out_shape=jax.ShapeDtypeStruct((N, DOUT), jnp.float32),
    )(parts, rootp)


def kernel(x, edge_index, edge_type, comp, bases, root, bias):
    bases2 = bases.reshape(NB, DIN * DOUT)
    w = _make_w(comp, bases2).reshape(R, DIN, DOUT)
    z, rootp = _make_z(x, w, root, bias.reshape(1, DOUT))
    zf = z.reshape(RN, DOUT)
    packed = ((edge_type << 28) | (edge_index[0] << 14) | edge_index[1])
    parts = _sc_agg(packed.astype(jnp.int32), zf)
    return _fin(parts, rootp)


# speedup vs baseline: 5.2604x; 1.1497x over previous
import functools

import jax
import jax.numpy as jnp
from jax import lax
from jax.experimental import pallas as pl
from jax.experimental.pallas import tpu as pltpu
from jax.experimental.pallas import tpu_sc as plsc

N = 10000
E = 320000
R = 16
NB = 8
DIN = 128
DOUT = 128

NC = 2
NS = 16
EPC = E // NC
EPT = EPC // NS
CEPT = E // NS
CH = 80
RN = R * N
RNT = RN // NS
NG = N // 8
IB = 2000


def _w_body(comp_ref, bases_ref, w_ref):
    w_ref[...] = jnp.dot(comp_ref[...], bases_ref[...],
                         preferred_element_type=jnp.float32)


def _make_w(comp, bases2):
    return pl.pallas_call(
        _w_body,
        out_shape=jax.ShapeDtypeStruct((R, DIN * DOUT), jnp.float32),
    )(comp, bases2)


BN = 1000
NBLK = N // BN


def _z_body(x_ref, w_ref, root_ref, bias_ref, z_ref, rp_ref):
    r = pl.program_id(1)
    xb = x_ref[...]
    z_ref[0] = jnp.dot(xb, w_ref[0], preferred_element_type=jnp.float32)

    @pl.when(r == 0)
    def _():
        rp_ref[...] = (jnp.dot(xb, root_ref[...],
                               preferred_element_type=jnp.float32)
                       + bias_ref[...])


def _make_z(x, w, root, bias2):
    return pl.pallas_call(
        _z_body,
        grid=(NBLK, R),
        in_specs=[
            pl.BlockSpec((BN, DIN), lambda nb, r: (nb, 0)),
            pl.BlockSpec((1, DIN, DOUT), lambda nb, r: (r, 0, 0)),
            pl.BlockSpec((DIN, DOUT), lambda nb, r: (0, 0)),
            pl.BlockSpec((1, DOUT), lambda nb, r: (0, 0)),
        ],
        out_specs=[
            pl.BlockSpec((1, BN, DOUT), lambda nb, r: (r, nb, 0)),
            pl.BlockSpec((BN, DOUT), lambda nb, r: (nb, 0)),
        ],
        out_shape=[
            jax.ShapeDtypeStruct((R, N, DOUT), jnp.float32),
            jax.ShapeDtypeStruct((N, DOUT), jnp.float32),
        ],
    )(x, w, root, bias2)


_sc_mesh = plsc.VectorSubcoreMesh(core_axis_name="c", subcore_axis_name="s")


@functools.partial(
    pl.kernel,
    out_type=jax.ShapeDtypeStruct((NC * RN,), jnp.float32),
    mesh=_sc_mesh,
    compiler_params=pltpu.CompilerParams(needs_layout_passes=False),
    scratch_types=[
        pltpu.VMEM_SHARED((RN,), jnp.float32),
        pltpu.VMEM((CH,), jnp.int32),
        pltpu.VMEM((CH,), jnp.int32),
        pltpu.VMEM((CH,), jnp.int32),
        pltpu.VMEM((CH,), jnp.int32),
        pltpu.VMEM((CH,), jnp.float32),
        pltpu.VMEM((IB,), jnp.float32),
        pltpu.SemaphoreType.DMA,
        pltpu.SemaphoreType.DMA,
        pltpu.SemaphoreType.DMA,
        pltpu.SemaphoreType.DMA,
    ],
)
def _sc_cnt(pk_hbm, out_hbm,
            cnt_sh, pb0, pb1, cb0, cb1, onesb, invb,
            sg0, sg1, ss0, ss1):
    cid = lax.axis_index("c")
    sid = lax.axis_index("s")
    m14 = jnp.full((16,), 16383, jnp.int32)

    def _z16(i, _):
        invb[pl.ds(i * 16, 16)] = jnp.zeros((16,), jnp.float32)
        return 0
    lax.fori_loop(0, IB // 16, _z16, 0)

    for i in range(CH // 16):
        onesb[pl.ds(i * 16, 16)] = jnp.ones((16,), jnp.float32)

    for h in range(RNT // IB):
        pltpu.sync_copy(invb, cnt_sh.at[pl.ds(sid * RNT + h * IB, IB)])
    plsc.subcore_barrier()

    ebase = cid * EPC + sid * EPT

    def _cload(c, pbp, sgp):
        pltpu.async_copy(pk_hbm.at[pl.ds(ebase + c * CH, CH)], pbp, sgp)

    def _ccount(c, pbp, sbp, sgp, ssp, wait_scatter):
        pltpu.make_async_copy(pk_hbm.at[pl.ds(0, CH)], pbp, sgp).wait()
        if wait_scatter:
            pltpu.make_async_copy(onesb, cnt_sh.at[sbp], ssp).wait()

        @plsc.parallel_loop(0, CH // 16, 1, unroll=5)
        def _mk(i):
            sl = pl.ds(i * 16, 16)
            w_ = pbp[sl]
            t = lax.shift_right_logical(w_, 28)
            sbp[sl] = t * N + (w_ & m14)
        pltpu.async_copy(onesb, cnt_sh.at[sbp], ssp, add=True)

    cslot0 = (pb0, cb0, sg0, ss0)
    cslot1 = (pb1, cb1, sg1, ss1)
    CNCH = EPT // CH

    _cload(0, pb0, sg0)
    _cload(1, pb1, sg1)
    _ccount(0, *cslot0, wait_scatter=False)
    _cload(2, pb0, sg0)
    _ccount(1, *cslot1, wait_scatter=False)
    _cload(3, pb1, sg1)

    def _cpipe(k, _):
        _ccount(2 * k + 2, *cslot0, wait_scatter=True)

        @pl.when(2 * k + 4 < CNCH)
        def _():
            _cload(2 * k + 4, pb0, sg0)
        _ccount(2 * k + 3, *cslot1, wait_scatter=True)

        @pl.when(2 * k + 5 < CNCH)
        def _():
            _cload(2 * k + 5, pb1, sg1)
        return 0
    lax.fori_loop(0, (CNCH - 2) // 2, _cpipe, 0)
    _ccount(CNCH - 1, *cslot0, wait_scatter=True)
    pltpu.make_async_copy(onesb, cnt_sh.at[cb0], ss0).wait()
    pltpu.make_async_copy(onesb, cnt_sh.at[cb1], ss1).wait()
    plsc.subcore_barrier()

    for h in range(RNT // IB):
        off = sid * RNT + h * IB
        pltpu.sync_copy(cnt_sh.at[pl.ds(off, IB)], invb)
        pltpu.sync_copy(invb, out_hbm.at[pl.ds(cid * RN + off, IB)])


def _inv_body(parts_ref, inv_ref):
    inv_ref[...] = 1.0 / jnp.maximum(parts_ref[0] + parts_ref[1], 1.0)


def _cnt_inv(parts):
    return pl.pallas_call(
        _inv_body,
        out_shape=jax.ShapeDtypeStruct((RN // DOUT, DOUT), jnp.float32),
    )(parts)


@functools.partial(
    pl.kernel,
    out_type=jax.ShapeDtypeStruct((NC, N, DOUT), jnp.float32),
    mesh=_sc_mesh,
    compiler_params=pltpu.CompilerParams(needs_layout_passes=False),
    scratch_types=[
        pltpu.VMEM_SHARED((N, DOUT), jnp.float32),
        pltpu.VMEM((CH,), jnp.int32),
        pltpu.VMEM((CH,), jnp.int32),
        pltpu.VMEM((CH,), jnp.int32),
        pltpu.VMEM((CH,), jnp.int32),
        pltpu.VMEM((CH,), jnp.int32),
        pltpu.VMEM((CH,), jnp.float32),
        pltpu.VMEM((CH,), jnp.float32),
        pltpu.VMEM((CH,), jnp.int32),
        pltpu.VMEM((CH,), jnp.int32),
        pltpu.VMEM((CH, DOUT), jnp.float32),
        pltpu.VMEM((CH, DOUT), jnp.float32),
        pltpu.SemaphoreType.DMA,
        pltpu.SemaphoreType.DMA,
        pltpu.SemaphoreType.DMA,
        pltpu.SemaphoreType.DMA,
        pltpu.SemaphoreType.DMA,
        pltpu.SemaphoreType.DMA,
    ],
)
def _sc_agg(pk_hbm, z_hbm, inv_hbm, out_hbm,
            agg_sh, pb,
            gb0, gb1, db0, db1, wb0, wb1, sb0, sb1,
            rows0, rows1, sg0, sg1, ss0, ss1, sw0, sw1):
    cid = lax.axis_index("c")
    sid = lax.axis_index("s")
    m14 = jnp.full((16,), 16383, jnp.int32)

    def _zrow(i, _):
        for k in range(DOUT // 16):
            rows0[i, pl.ds(k * 16, 16)] = jnp.zeros((16,), jnp.float32)
        return 0
    lax.fori_loop(0, CH, _zrow, 0)

    def _zagg(k, _):
        g = k * NS + sid

        @pl.when(g < NG)
        def _():
            pltpu.sync_copy(rows0.at[pl.ds(0, 8)],
                            agg_sh.at[pl.ds(g * 8, 8)])
        return 0
    lax.fori_loop(0, (NG + NS - 1) // NS, _zagg, 0)
    plsc.subcore_barrier()

    ebase = cid * EPC + sid * EPT

    def _prep_fire(c, gbp, dbp, wbp, sbp, rowsp, sgp, ssp, swp, wait_scatter):
        if wait_scatter:
            pltpu.make_async_copy(rowsp, agg_sh.at[dbp], ssp).wait()

        pltpu.sync_copy(pk_hbm.at[pl.ds(ebase + c * CH, CH)], pb)

        @plsc.parallel_loop(0, CH // 16, 1, unroll=5)
        def _mk(i):
            sl = pl.ds(i * 16, 16)
            w_ = pb[sl]
            t = lax.shift_right_logical(w_, 28)
            d = w_ & m14
            gbp[sl] = t * N + (lax.shift_right_logical(w_, 14) & m14)
            dbp[sl] = d
            sbp[sl] = t * N + d
        pltpu.async_copy(inv_hbm.at[sbp], wbp, swp)
        pltpu.async_copy(z_hbm.at[gbp], rowsp, sgp)

    def _consume(gbp, dbp, wbp, sbp, rowsp, sgp, ssp, swp):
        pltpu.make_async_copy(inv_hbm.at[sbp], wbp, swp).wait()
        pltpu.make_async_copy(z_hbm.at[gbp], rowsp, sgp).wait()

        @plsc.parallel_loop(0, CH, 1, unroll=4)
        def _scale(j):
            wsp = plsc.load_gather(wbp, [jnp.full((16,), j, jnp.int32)])
            for k in range(DOUT // 16):
                sl = pl.ds(k * 16, 16)
                rowsp[j, sl] = rowsp[j, sl] * wsp
        pltpu.async_copy(rowsp, agg_sh.at[dbp], ssp, add=True)

    slot0 = (gb0, db0, wb0, sb0, rows0, sg0, ss0, sw0)
    slot1 = (gb1, db1, wb1, sb1, rows1, sg1, ss1, sw1)
    NCH = EPT // CH

    _prep_fire(0, *slot0, wait_scatter=False)
    _prep_fire(1, *slot1, wait_scatter=False)

    def _pipe(k, _):
        _consume(*slot0)
        _prep_fire(2 * k + 2, *slot0, wait_scatter=True)
        _consume(*slot1)

        @pl.when(2 * k + 3 < NCH)
        def _():
            _prep_fire(2 * k + 3, *slot1, wait_scatter=True)
        return 0
    lax.fori_loop(0, (NCH - 1) // 2, _pipe, 0)
    _consume(*slot0)
    pltpu.make_async_copy(rows0, agg_sh.at[db0], ss0).wait()
    pltpu.make_async_copy(rows1, agg_sh.at[db1], ss1).wait()
    plsc.subcore_barrier()

    def _wout(k, _):
        g = k * NS + sid

        @pl.when(g < NG)
        def _():
            pltpu.sync_copy(agg_sh.at[pl.ds(g * 8, 8)],
                            out_hbm.at[cid, pl.ds(g * 8, 8)])
        return 0
    lax.fori_loop(0, (NG + NS - 1) // NS, _wout, 0)


def _fin_body(parts_ref, rp_ref, out_ref):
    out_ref[...] = parts_ref[0] + parts_ref[1] + rp_ref[...]


def _fin(parts, rootp):
    return pl.pallas_call(
        _fin_body,
        grid=(NBLK,),
        in_specs=[
            pl.BlockSpec((NC, BN, DOUT), lambda nb: (0, nb, 0)),
            pl.BlockSpec((BN, DOUT), lambda nb: (nb, 0)),
        ],
        out_specs=pl.BlockSpec((BN, DOUT), lambda nb: (nb, 0)),
        out_shape=jax.ShapeDtypeStruct((N, DOUT), jnp.float32),
    )(parts, rootp)


def kernel(x, edge_index, edge_type, comp, bases, root, bias):
    bases2 = bases.reshape(NB, DIN * DOUT)
    w = _make_w(comp, bases2).reshape(R, DIN, DOUT)
    packed = ((edge_type << 28) | (edge_index[0] << 14) | edge_index[1])
    packed = packed.astype(jnp.int32)
    cnt_parts = _sc_cnt(packed)
    z, rootp = _make_z(x, w, root, bias.reshape(1, DOUT))
    inv = _cnt_inv(cnt_parts.reshape(NC, RN // DOUT, DOUT))
    parts = _sc_agg(packed, z.reshape(RN, DOUT), inv.reshape(RN))
    return _fin(parts, rootp)


# speedup vs baseline: 5.7977x; 1.1021x over previous
import functools

import jax
import jax.numpy as jnp
from jax import lax
from jax.experimental import pallas as pl
from jax.experimental.pallas import tpu as pltpu
from jax.experimental.pallas import tpu_sc as plsc

N = 10000
E = 320000
R = 16
NB = 8
DIN = 128
DOUT = 128

NC = 2
NS = 16
EPC = E // NC
EPT = EPC // NS
CEPT = E // NS
CH = 80
RN = R * N
RNT = RN // NS
NG = N // 8
IB = 2000


def _w_body(comp_ref, bases_ref, w_ref):
    w_ref[...] = jnp.dot(comp_ref[...], bases_ref[...],
                         preferred_element_type=jnp.float32)


def _make_w(comp, bases2):
    return pl.pallas_call(
        _w_body,
        out_shape=jax.ShapeDtypeStruct((R, DIN * DOUT), jnp.float32),
    )(comp, bases2)


BN = 1000
NBLK = N // BN


def _z_body(x_ref, w_ref, root_ref, bias_ref, z_ref, rp_ref):
    r = pl.program_id(1)
    xb = x_ref[...]
    z_ref[0] = jnp.dot(xb, w_ref[0], preferred_element_type=jnp.float32)

    @pl.when(r == 0)
    def _():
        rp_ref[...] = (jnp.dot(xb, root_ref[...],
                               preferred_element_type=jnp.float32)
                       + bias_ref[...])


def _make_z(x, w, root, bias2):
    return pl.pallas_call(
        _z_body,
        grid=(NBLK, R),
        in_specs=[
            pl.BlockSpec((BN, DIN), lambda nb, r: (nb, 0)),
            pl.BlockSpec((1, DIN, DOUT), lambda nb, r: (r, 0, 0)),
            pl.BlockSpec((DIN, DOUT), lambda nb, r: (0, 0)),
            pl.BlockSpec((1, DOUT), lambda nb, r: (0, 0)),
        ],
        out_specs=[
            pl.BlockSpec((1, BN, DOUT), lambda nb, r: (r, nb, 0)),
            pl.BlockSpec((BN, DOUT), lambda nb, r: (nb, 0)),
        ],
        out_shape=[
            jax.ShapeDtypeStruct((R, N, DOUT), jnp.float32),
            jax.ShapeDtypeStruct((N, DOUT), jnp.float32),
        ],
    )(x, w, root, bias2)


_sc_mesh = plsc.VectorSubcoreMesh(core_axis_name="c", subcore_axis_name="s")


@functools.partial(
    pl.kernel,
    out_type=jax.ShapeDtypeStruct((NC * RN,), jnp.float32),
    mesh=_sc_mesh,
    compiler_params=pltpu.CompilerParams(needs_layout_passes=False),
    scratch_types=[
        pltpu.VMEM_SHARED((RN,), jnp.float32),
        pltpu.VMEM((CH,), jnp.int32),
        pltpu.VMEM((CH,), jnp.int32),
        pltpu.VMEM((CH,), jnp.int32),
        pltpu.VMEM((CH,), jnp.int32),
        pltpu.VMEM((CH,), jnp.float32),
        pltpu.VMEM((IB,), jnp.float32),
        pltpu.SemaphoreType.DMA,
        pltpu.SemaphoreType.DMA,
        pltpu.SemaphoreType.DMA,
        pltpu.SemaphoreType.DMA,
    ],
)
def _sc_cnt(pk_hbm, out_hbm,
            cnt_sh, pb0, pb1, cb0, cb1, onesb, invb,
            sg0, sg1, ss0, ss1):
    cid = lax.axis_index("c")
    sid = lax.axis_index("s")
    m14 = jnp.full((16,), 16383, jnp.int32)

    def _z16(i, _):
        invb[pl.ds(i * 16, 16)] = jnp.zeros((16,), jnp.float32)
        return 0
    lax.fori_loop(0, IB // 16, _z16, 0)

    for i in range(CH // 16):
        onesb[pl.ds(i * 16, 16)] = jnp.ones((16,), jnp.float32)

    for h in range(RNT // IB):
        pltpu.sync_copy(invb, cnt_sh.at[pl.ds(sid * RNT + h * IB, IB)])
    plsc.subcore_barrier()

    ebase = cid * EPC + sid * EPT

    def _cload(c, pbp, sgp):
        pltpu.async_copy(pk_hbm.at[pl.ds(ebase + c * CH, CH)], pbp, sgp)

    def _ccount(c, pbp, sbp, sgp, ssp, wait_scatter):
        pltpu.make_async_copy(pk_hbm.at[pl.ds(0, CH)], pbp, sgp).wait()
        if wait_scatter:
            pltpu.make_async_copy(onesb, cnt_sh.at[sbp], ssp).wait()

        @plsc.parallel_loop(0, CH // 16, 1, unroll=5)
        def _mk(i):
            sl = pl.ds(i * 16, 16)
            w_ = pbp[sl]
            t = lax.shift_right_logical(w_, 28)
            sbp[sl] = t * N + (w_ & m14)
        pltpu.async_copy(onesb, cnt_sh.at[sbp], ssp, add=True)

    cslot0 = (pb0, cb0, sg0, ss0)
    cslot1 = (pb1, cb1, sg1, ss1)
    CNCH = EPT // CH

    _cload(0, pb0, sg0)
    _cload(1, pb1, sg1)
    _ccount(0, *cslot0, wait_scatter=False)
    _cload(2, pb0, sg0)
    _ccount(1, *cslot1, wait_scatter=False)
    _cload(3, pb1, sg1)

    def _cpipe(k, _):
        _ccount(2 * k + 2, *cslot0, wait_scatter=True)

        @pl.when(2 * k + 4 < CNCH)
        def _():
            _cload(2 * k + 4, pb0, sg0)
        _ccount(2 * k + 3, *cslot1, wait_scatter=True)

        @pl.when(2 * k + 5 < CNCH)
        def _():
            _cload(2 * k + 5, pb1, sg1)
        return 0
    lax.fori_loop(0, (CNCH - 2) // 2, _cpipe, 0)
    _ccount(CNCH - 1, *cslot0, wait_scatter=True)
    pltpu.make_async_copy(onesb, cnt_sh.at[cb0], ss0).wait()
    pltpu.make_async_copy(onesb, cnt_sh.at[cb1], ss1).wait()
    plsc.subcore_barrier()

    for h in range(RNT // IB):
        off = sid * RNT + h * IB
        pltpu.sync_copy(cnt_sh.at[pl.ds(off, IB)], invb)
        pltpu.sync_copy(invb, out_hbm.at[pl.ds(cid * RN + off, IB)])


def _inv_body(parts_ref, inv_ref):
    inv_ref[...] = 1.0 / jnp.maximum(parts_ref[0] + parts_ref[1], 1.0)


def _cnt_inv(parts):
    return pl.pallas_call(
        _inv_body,
        out_shape=jax.ShapeDtypeStruct((RN // DOUT, DOUT), jnp.float32),
    )(parts)


@functools.partial(
    pl.kernel,
    out_type=jax.ShapeDtypeStruct((NC, N, DOUT), jnp.float32),
    mesh=_sc_mesh,
    compiler_params=pltpu.CompilerParams(needs_layout_passes=False),
    scratch_types=[
        pltpu.VMEM_SHARED((N, DOUT), jnp.float32),
        pltpu.VMEM((CH,), jnp.int32),
        pltpu.VMEM((CH,), jnp.int32),
        pltpu.VMEM((CH,), jnp.int32),
        pltpu.VMEM((CH,), jnp.int32),
        pltpu.VMEM((CH,), jnp.int32),
        pltpu.VMEM((CH,), jnp.int32),
        pltpu.VMEM((CH,), jnp.float32),
        pltpu.VMEM((CH,), jnp.float32),
        pltpu.VMEM((CH,), jnp.int32),
        pltpu.VMEM((CH,), jnp.int32),
        pltpu.VMEM((CH, DOUT), jnp.float32),
        pltpu.VMEM((CH, DOUT), jnp.float32),
        pltpu.SemaphoreType.DMA,
        pltpu.SemaphoreType.DMA,
        pltpu.SemaphoreType.DMA,
        pltpu.SemaphoreType.DMA,
        pltpu.SemaphoreType.DMA,
        pltpu.SemaphoreType.DMA,
        pltpu.SemaphoreType.DMA,
        pltpu.SemaphoreType.DMA,
    ],
)
def _sc_agg(pk_hbm, z_hbm, inv_hbm, out_hbm,
            agg_sh, pb0, pb1,
            gb0, gb1, db0, db1, wb0, wb1, sb0, sb1,
            rows0, rows1, sg0, sg1, ss0, ss1, sw0, sw1, sp0, sp1):
    cid = lax.axis_index("c")
    sid = lax.axis_index("s")
    m14 = jnp.full((16,), 16383, jnp.int32)

    def _zrow(i, _):
        for k in range(DOUT // 16):
            rows0[i, pl.ds(k * 16, 16)] = jnp.zeros((16,), jnp.float32)
        return 0
    lax.fori_loop(0, CH, _zrow, 0)

    def _zagg(k, _):
        g = k * NS + sid

        @pl.when(g < NG)
        def _():
            pltpu.sync_copy(rows0.at[pl.ds(0, 8)],
                            agg_sh.at[pl.ds(g * 8, 8)])
        return 0
    lax.fori_loop(0, (NG + NS - 1) // NS, _zagg, 0)
    plsc.subcore_barrier()

    ebase = cid * EPC + sid * EPT

    def _prep_fire(c, pbp, gbp, dbp, wbp, sbp, rowsp, sgp, ssp, swp, spp,
                   wait_scatter):
        if wait_scatter:
            pltpu.make_async_copy(rowsp, agg_sh.at[dbp], ssp).wait()

        pltpu.make_async_copy(pk_hbm.at[pl.ds(0, CH)], pbp, spp).wait()

        @plsc.parallel_loop(0, CH // 16, 1, unroll=5)
        def _mk(i):
            sl = pl.ds(i * 16, 16)
            w_ = pbp[sl]
            t = lax.shift_right_logical(w_, 28)
            d = w_ & m14
            gbp[sl] = t * N + (lax.shift_right_logical(w_, 14) & m14)
            dbp[sl] = d
            sbp[sl] = t * N + d
        pltpu.async_copy(inv_hbm.at[sbp], wbp, swp)
        pltpu.async_copy(z_hbm.at[gbp], rowsp, sgp)

        @pl.when(c + 2 < NCH)
        def _():
            pltpu.async_copy(
                pk_hbm.at[pl.ds(ebase + (c + 2) * CH, CH)], pbp, spp)

    def _consume(pbp, gbp, dbp, wbp, sbp, rowsp, sgp, ssp, swp, spp):
        del pbp, spp
        pltpu.make_async_copy(inv_hbm.at[sbp], wbp, swp).wait()
        pltpu.make_async_copy(z_hbm.at[gbp], rowsp, sgp).wait()

        @plsc.parallel_loop(0, CH, 1, unroll=4)
        def _scale(j):
            wsp = plsc.load_gather(wbp, [jnp.full((16,), j, jnp.int32)])
            for k in range(DOUT // 16):
                sl = pl.ds(k * 16, 16)
                rowsp[j, sl] = rowsp[j, sl] * wsp
        pltpu.async_copy(rowsp, agg_sh.at[dbp], ssp, add=True)

    slot0 = (pb0, gb0, db0, wb0, sb0, rows0, sg0, ss0, sw0, sp0)
    slot1 = (pb1, gb1, db1, wb1, sb1, rows1, sg1, ss1, sw1, sp1)
    NCH = EPT // CH

    pltpu.async_copy(pk_hbm.at[pl.ds(ebase, CH)], pb0, sp0)
    pltpu.async_copy(pk_hbm.at[pl.ds(ebase + CH, CH)], pb1, sp1)
    _prep_fire(0, *slot0, wait_scatter=False)
    _prep_fire(1, *slot1, wait_scatter=False)

    def _pipe(k, _):
        _consume(*slot0)
        _prep_fire(2 * k + 2, *slot0, wait_scatter=True)
        _consume(*slot1)

        @pl.when(2 * k + 3 < NCH)
        def _():
            _prep_fire(2 * k + 3, *slot1, wait_scatter=True)
        return 0
    lax.fori_loop(0, (NCH - 1) // 2, _pipe, 0)
    _consume(*slot0)
    pltpu.make_async_copy(rows0, agg_sh.at[db0], ss0).wait()
    pltpu.make_async_copy(rows1, agg_sh.at[db1], ss1).wait()
    plsc.subcore_barrier()

    def _wout(k, _):
        g = k * NS + sid

        @pl.when(g < NG)
        def _():
            pltpu.sync_copy(agg_sh.at[pl.ds(g * 8, 8)],
                            out_hbm.at[cid, pl.ds(g * 8, 8)])
        return 0
    lax.fori_loop(0, (NG + NS - 1) // NS, _wout, 0)


def _fin_body(parts_ref, rp_ref, out_ref):
    out_ref[...] = parts_ref[0] + parts_ref[1] + rp_ref[...]


def _fin(parts, rootp):
    return pl.pallas_call(
        _fin_body,
        grid=(NBLK,),
        in_specs=[
            pl.BlockSpec((NC, BN, DOUT), lambda nb: (0, nb, 0)),
            pl.BlockSpec((BN, DOUT), lambda nb: (nb, 0)),
        ],
        out_specs=pl.BlockSpec((BN, DOUT), lambda nb: (nb, 0)),
        out_shape=jax.ShapeDtypeStruct((N, DOUT), jnp.float32),
    )(parts, rootp)


def kernel(x, edge_index, edge_type, comp, bases, root, bias):
    bases2 = bases.reshape(NB, DIN * DOUT)
    w = _make_w(comp, bases2).reshape(R, DIN, DOUT)
    packed = ((edge_type << 28) | (edge_index[0] << 14) | edge_index[1])
    packed = packed.astype(jnp.int32)
    cnt_parts = _sc_cnt(packed)
    z, rootp = _make_z(x, w, root, bias.reshape(1, DOUT))
    inv = _cnt_inv(cnt_parts.reshape(NC, RN // DOUT, DOUT))
    parts = _sc_agg(packed, z.reshape(RN, DOUT), inv.reshape(RN))
    return _fin(parts, rootp)
